# baseline (speedup 1.0000x reference)
import functools

import jax
import jax.numpy as jnp
from jax import lax
from jax.experimental import pallas as pl
from jax.experimental.pallas import tpu as pltpu
from jax.experimental.pallas import tpu_sc as plsc

_WINDOW = 200
_NUM_CORES = 1
_NUM_SUBCORES = 16
_NW = _NUM_CORES * _NUM_SUBCORES
_LANES = 16
_SHIFTS = 8


@functools.partial(jax.jit, static_argnums=(2, 3, 4))
def _sc_window_gather(tables, start, batch, channels, b_per_w):
    rows_ext = tables.shape[1]
    mesh = plsc.VectorSubcoreMesh(
        core_axis_name="c",
        subcore_axis_name="s",
        num_cores=_NUM_CORES,
        num_subcores=_NUM_SUBCORES,
    )

    @functools.partial(
        pl.kernel,
        mesh=mesh,
        out_type=jax.ShapeDtypeStruct((batch, _WINDOW, channels), jnp.float32),
        scratch_types=[
            pltpu.VMEM((b_per_w,), jnp.int32),
            pltpu.VMEM_SHARED((_SHIFTS, rows_ext, channels), jnp.float32),
            pltpu.SemaphoreType.DMA,
            pltpu.SemaphoreType.DMA,
        ],
        compiler_params=pltpu.CompilerParams(use_tc_tiling_on_sc=True),
    )
    def k(tbl_hbm, start_hbm, out_hbm, idx_v, tbl_sp, sem_idx, sem_out):
        sid = lax.axis_index("s")
        wid = sid * _NUM_CORES + lax.axis_index("c")
        base = wid * b_per_w
        idx_cp = pltpu.make_async_copy(
            start_hbm.at[pl.ds(base, b_per_w)], idx_v, sem_idx
        )
        idx_cp.start()

        @pl.when(sid == 0)
        def _():
            pltpu.make_async_copy(tbl_hbm, tbl_sp, sem_out).start()
            pltpu.make_async_copy(tbl_hbm, tbl_sp, sem_out).wait()

        idx_cp.wait()
        plsc.subcore_barrier()

        def fire(g, carry):
            vec = idx_v[pl.ds(g * _LANES, _LANES)]
            shift = lax.rem(vec, _SHIFTS)
            aligned = vec - shift
            for lane in range(_LANES):
                pltpu.make_async_copy(
                    tbl_sp.at[
                        shift[lane],
                        pl.ds(pl.multiple_of(aligned[lane], _SHIFTS), _WINDOW),
                    ],
                    out_hbm.at[base + g * _LANES + lane],
                    sem_out,
                ).start()
            return carry

        lax.fori_loop(0, b_per_w // _LANES, fire, 0)

        def drain(b, carry):
            pltpu.make_async_copy(
                tbl_sp.at[0, pl.ds(0, _WINDOW)], out_hbm.at[base + b], sem_out
            ).wait()
            return carry

        lax.fori_loop(0, b_per_w, drain, 0)

    return k(tables, start)


def kernel(index, length, data):
    cycle_len, channels = data.shape
    batch = index.shape[0]
    start = jnp.asarray(
        (index.astype(jnp.int32) + (length - _WINDOW)) % cycle_len, jnp.int32
    )
    rows_ext = cycle_len + _WINDOW
    data_ext = jnp.concatenate([data, data[: _WINDOW + _SHIFTS]], axis=0)
    tables = jnp.stack([data_ext[k : k + rows_ext] for k in range(_SHIFTS)])
    return _sc_window_gather(tables, start, batch, channels, batch // _NW)


# speedup vs baseline: 1.2540x; 1.2540x over previous
import functools

import jax
import jax.numpy as jnp
from jax import lax
from jax.experimental import pallas as pl
from jax.experimental.pallas import tpu as pltpu
from jax.experimental.pallas import tpu_sc as plsc

_WINDOW = 200
_NUM_CORES = 2
_NUM_SUBCORES = 16
_NW = _NUM_CORES * _NUM_SUBCORES
_LANES = 16
_BB = 128


@functools.partial(jax.jit, static_argnums=(2, 3, 4))
def _sc_window_gather(table_flat, start, batch, channels, b_per_w):
    ext_words = table_flat.shape[0]
    out_words = _WINDOW * channels
    mesh = plsc.VectorSubcoreMesh(
        core_axis_name="c",
        subcore_axis_name="s",
        num_cores=_NUM_CORES,
        num_subcores=_NUM_SUBCORES,
    )

    @functools.partial(
        pl.kernel,
        mesh=mesh,
        out_type=jax.ShapeDtypeStruct((batch * out_words,), jnp.float32),
        scratch_types=[
            pltpu.VMEM((b_per_w,), jnp.int32),
            pltpu.VMEM((ext_words,), jnp.float32),
            pltpu.SemaphoreType.DMA,
            pltpu.SemaphoreType.DMA,
        ],
        compiler_params=pltpu.CompilerParams(use_tc_tiling_on_sc=False),
    )
    def k(table_hbm, start_hbm, out_hbm, idx_v, table_v, sem_idx, sem_out):
        wid = lax.axis_index("s") * _NUM_CORES + lax.axis_index("c")
        base = wid * b_per_w
        idx_cp = pltpu.make_async_copy(
            start_hbm.at[pl.ds(base, b_per_w)], idx_v, sem_idx
        )
        tbl_cp = pltpu.make_async_copy(table_hbm, table_v, sem_out)
        idx_cp.start()
        tbl_cp.start()
        idx_cp.wait()
        tbl_cp.wait()

        def fire(g, carry):
            vec = idx_v[pl.ds(g * _LANES, _LANES)] * channels
            for lane in range(_LANES):
                b = base + g * _LANES + lane
                pltpu.make_async_copy(
                    table_v.at[pl.ds(pl.multiple_of(vec[lane], 8), out_words)],
                    out_hbm.at[pl.ds(pl.multiple_of(b * out_words, 8), out_words)],
                    sem_out,
                ).start()
            return carry

        lax.fori_loop(0, b_per_w // _LANES, fire, 0)

        def drain(b, carry):
            pltpu.make_async_copy(
                table_v.at[pl.ds(0, out_words)],
                out_hbm.at[pl.ds(pl.multiple_of((base + b) * out_words, 8), out_words)],
                sem_out,
            ).wait()
            return carry

        lax.fori_loop(0, b_per_w, drain, 0)

    return k(table_flat, start)


@functools.partial(jax.jit, static_argnums=(1, 2))
def _tc_transpose(lin, batch, out_words):
    def body(in_hbm, out_ref, scratch, sem):
        i = pl.program_id(0)

        def ld(b, carry):
            pltpu.make_async_copy(
                in_hbm.at[
                    pl.ds(pl.multiple_of((i * _BB + b) * out_words, 8), out_words)
                ],
                scratch.at[b],
                sem,
            ).start()
            return carry

        lax.fori_loop(0, _BB, ld, 0)

        def wt(b, carry):
            pltpu.make_async_copy(
                in_hbm.at[pl.ds(0, out_words)], scratch.at[b], sem
            ).wait()
            return carry

        lax.fori_loop(0, _BB, wt, 0)

        def tp(k, carry):
            out_ref[pl.ds(k * _BB, _BB), :] = scratch[:, pl.ds(k * _BB, _BB)].T
            return carry

        lax.fori_loop(0, out_words // _BB, tp, 0)

    return pl.pallas_call(
        body,
        grid=(batch // _BB,),
        in_specs=[pl.BlockSpec(memory_space=pl.ANY)],
        out_specs=pl.BlockSpec((out_words, _BB), lambda i: (0, i)),
        out_shape=jax.ShapeDtypeStruct((out_words, batch), jnp.float32),
        scratch_shapes=[
            pltpu.VMEM((_BB, out_words), jnp.float32),
            pltpu.SemaphoreType.DMA,
        ],
    )(lin)


def kernel(index, length, data):
    cycle_len, channels = data.shape
    batch = index.shape[0]
    start = jnp.asarray(
        (index.astype(jnp.int32) + (length - _WINDOW)) % cycle_len, jnp.int32
    )
    table_flat = jnp.concatenate([data, data[:_WINDOW]], axis=0).reshape(-1)
    out_words = _WINDOW * channels
    lin = _sc_window_gather(table_flat, start, batch, channels, batch // _NW)
    outT = _tc_transpose(lin, batch, out_words)
    return outT.reshape(_WINDOW, channels, batch).transpose(2, 0, 1)


# speedup vs baseline: 2.7782x; 2.2155x over previous
import functools

import jax
import jax.numpy as jnp
from jax import lax
from jax.experimental import pallas as pl
from jax.experimental.pallas import tpu as pltpu
from jax.experimental.pallas import tpu_sc as plsc

_WINDOW = 200
_NUM_CORES = 2
_NUM_SUBCORES = 16
_NW = _NUM_CORES * _NUM_SUBCORES
_LANES = 16
_BB = 128


@functools.partial(jax.jit, static_argnums=(2, 3))
def _sc_window_gather(tbl2, start, batch, b_per_w):
    _, half_rows, lanes = tbl2.shape
    kchunks = _WINDOW * 64 // lanes
    mesh = plsc.VectorSubcoreMesh(
        core_axis_name="c",
        subcore_axis_name="s",
        num_cores=_NUM_CORES,
        num_subcores=_NUM_SUBCORES,
    )

    @functools.partial(
        pl.kernel,
        mesh=mesh,
        out_type=jax.ShapeDtypeStruct((kchunks, batch, lanes), jnp.float32),
        scratch_types=[
            pltpu.VMEM((b_per_w,), jnp.int32),
            pltpu.VMEM_SHARED((2, half_rows, lanes), jnp.float32),
            pltpu.SemaphoreType.DMA,
            pltpu.SemaphoreType.DMA,
        ],
        compiler_params=pltpu.CompilerParams(use_tc_tiling_on_sc=False),
    )
    def k(tbl_hbm, start_hbm, out_hbm, idx_v, tbl_sp, sem_idx, sem_out):
        sid = lax.axis_index("s")
        wid = sid * _NUM_CORES + lax.axis_index("c")
        base = wid * b_per_w
        idx_cp = pltpu.make_async_copy(
            start_hbm.at[pl.ds(base, b_per_w)], idx_v, sem_idx
        )
        idx_cp.start()

        @pl.when(sid == 0)
        def _():
            pltpu.make_async_copy(tbl_hbm, tbl_sp, sem_out).start()
            pltpu.make_async_copy(tbl_hbm, tbl_sp, sem_out).wait()

        idx_cp.wait()
        plsc.subcore_barrier()

        def fire(g, carry):
            vec = idx_v[pl.ds(g * _LANES, _LANES)]
            parity = lax.rem(vec, 2)
            row = lax.shift_right_logical(vec, 1)
            for lane in range(_LANES):
                pltpu.make_async_copy(
                    tbl_sp.at[parity[lane], pl.ds(row[lane], kchunks), :],
                    out_hbm.at[:, base + g * _LANES + lane, :],
                    sem_out,
                ).start()
            return carry

        lax.fori_loop(0, b_per_w // _LANES, fire, 0)

        def drain(b, carry):
            pltpu.make_async_copy(
                tbl_sp.at[0, pl.ds(0, kchunks), :],
                out_hbm.at[:, base + b, :],
                sem_out,
            ).wait()
            return carry

        lax.fori_loop(0, b_per_w, drain, 0)

    return k(tbl2, start)


@functools.partial(jax.jit, static_argnums=(1,))
def _tc_transpose(m, batch):
    kchunks = m.shape[0]

    def body(in_ref, out_ref):
        for k in range(kchunks):
            out_ref[k * _BB : (k + 1) * _BB, :] = in_ref[k].T

    return pl.pallas_call(
        body,
        grid=(batch // _BB,),
        in_specs=[pl.BlockSpec((kchunks, _BB, _BB), lambda i: (0, i, 0))],
        out_specs=pl.BlockSpec((kchunks * _BB, _BB), lambda i: (0, i)),
        out_shape=jax.ShapeDtypeStruct((kchunks * _BB, batch), jnp.float32),
    )(m)


def kernel(index, length, data):
    cycle_len, channels = data.shape
    batch = index.shape[0]
    start = jnp.asarray(
        (index.astype(jnp.int32) + (length - _WINDOW)) % cycle_len, jnp.int32
    )
    flat = jnp.concatenate([data, data[: _WINDOW + 1]], axis=0).reshape(-1)
    half_words = (cycle_len // 2 + _WINDOW // 2) * 2 * channels
    tbl2 = jnp.stack(
        [
            flat[:half_words].reshape(-1, 2 * channels),
            flat[channels : half_words + channels].reshape(-1, 2 * channels),
        ]
    )
    m = _sc_window_gather(tbl2, start, batch, batch // _NW)
    outT = _tc_transpose(m, batch)
    return outT.reshape(_WINDOW, channels, batch).transpose(2, 0, 1)


# speedup vs baseline: 2.8967x; 1.0427x over previous
import functools

import jax
import jax.numpy as jnp
from jax import lax
from jax.experimental import pallas as pl
from jax.experimental.pallas import tpu as pltpu
from jax.experimental.pallas import tpu_sc as plsc

_WINDOW = 200
_NUM_CORES = 2
_NUM_SUBCORES = 16
_NW = _NUM_CORES * _NUM_SUBCORES
_LANES = 16
_BB = 128


@functools.partial(jax.jit, static_argnums=(2, 3, 4, 5))
def _sc_window_gather(tbl2, start, batch, b_per_w, phase, nphases):
    _, half_rows, lanes = tbl2.shape
    kchunks = _WINDOW * 64 // lanes // nphases
    mesh = plsc.VectorSubcoreMesh(
        core_axis_name="c",
        subcore_axis_name="s",
        num_cores=_NUM_CORES,
        num_subcores=_NUM_SUBCORES,
    )

    @functools.partial(
        pl.kernel,
        mesh=mesh,
        out_type=jax.ShapeDtypeStruct((kchunks, batch, lanes), jnp.float32),
        scratch_types=[
            pltpu.VMEM((b_per_w,), jnp.int32),
            pltpu.VMEM_SHARED((2, half_rows, lanes), jnp.float32),
            pltpu.SemaphoreType.DMA,
            pltpu.SemaphoreType.DMA,
        ],
        compiler_params=pltpu.CompilerParams(use_tc_tiling_on_sc=False),
    )
    def k(tbl_hbm, start_hbm, out_hbm, idx_v, tbl_sp, sem_idx, sem_out):
        sid = lax.axis_index("s")
        wid = sid * _NUM_CORES + lax.axis_index("c")
        base = wid * b_per_w
        idx_cp = pltpu.make_async_copy(
            start_hbm.at[pl.ds(base, b_per_w)], idx_v, sem_idx
        )
        idx_cp.start()

        @pl.when(sid == 0)
        def _():
            pltpu.make_async_copy(tbl_hbm, tbl_sp, sem_out).start()
            pltpu.make_async_copy(tbl_hbm, tbl_sp, sem_out).wait()

        idx_cp.wait()
        plsc.subcore_barrier()

        def fire(g, carry):
            vec = idx_v[pl.ds(g * _LANES, _LANES)]
            parity = lax.rem(vec, 2)
            row = lax.shift_right_logical(vec, 1) + phase * kchunks
            for lane in range(_LANES):
                pltpu.make_async_copy(
                    tbl_sp.at[parity[lane], pl.ds(row[lane], kchunks), :],
                    out_hbm.at[:, base + g * _LANES + lane, :],
                    sem_out,
                ).start()
            return carry

        lax.fori_loop(0, b_per_w // _LANES, fire, 0)

        def drain(b, carry):
            pltpu.make_async_copy(
                tbl_sp.at[0, pl.ds(0, kchunks), :],
                out_hbm.at[:, base + b, :],
                sem_out,
            ).wait()
            return carry

        lax.fori_loop(0, b_per_w, drain, 0)

    return k(tbl2, start)


@functools.partial(jax.jit, static_argnums=(2, 3, 4))
def _tc_transpose(m, prev, batch, phase, nphases):
    kchunks = m.shape[0]
    rows = kchunks * _BB * nphases

    def body(*refs):
        in_ref, out_ref = refs[-2], refs[-1]
        for k in range(kchunks):
            out_ref[k * _BB : (k + 1) * _BB, :] = in_ref[k].T

    m_spec = pl.BlockSpec((kchunks, _BB, _BB), lambda i: (0, i, 0))
    operands = (m,) if prev is None else (prev, m)
    in_specs = [m_spec] if prev is None else [
        pl.BlockSpec(memory_space=pl.ANY),
        m_spec,
    ]
    return pl.pallas_call(
        body,
        grid=(batch // _BB,),
        in_specs=in_specs,
        out_specs=pl.BlockSpec(
            (kchunks * _BB, _BB), lambda i, _p=phase: (_p, i)
        ),
        out_shape=jax.ShapeDtypeStruct((rows, batch), jnp.float32),
        input_output_aliases={} if prev is None else {0: 0},
    )(*operands)


def kernel(index, length, data):
    cycle_len, channels = data.shape
    batch = index.shape[0]
    start = jnp.asarray(
        (index.astype(jnp.int32) + (length - _WINDOW)) % cycle_len, jnp.int32
    )
    flat = jnp.concatenate([data, data[: _WINDOW + 1]], axis=0).reshape(-1)
    half_words = (cycle_len // 2 + _WINDOW // 2) * 2 * channels
    tbl2 = jnp.stack(
        [
            flat[:half_words].reshape(-1, 2 * channels),
            flat[channels : half_words + channels].reshape(-1, 2 * channels),
        ]
    )
    nphases = 2
    outT = None
    for phase in range(nphases):
        m = _sc_window_gather(tbl2, start, batch, batch // _NW, phase, nphases)
        outT = _tc_transpose(m, outT, batch, phase, nphases)
    return outT.reshape(_WINDOW, channels, batch).transpose(2, 0, 1)
